# baseline (speedup 1.0000x reference)
import functools

import jax
import jax.numpy as jnp
from jax import lax
from jax.experimental import pallas as pl
from jax.experimental.pallas import tpu as pltpu
from jax.experimental.pallas import tpu_sc as plsc

def _lane_take(v, idx):
    dnums = lax.GatherDimensionNumbers(
        offset_dims=(), collapsed_slice_dims=(0,), start_index_map=(0,))
    return lax.gather(v, idx[:, None], dnums, slice_sizes=(1,),
                      mode=lax.GatherScatterMode.PROMISE_IN_BOUNDS)


N_NODES = 10000
N_EDGES = 320000
D = 128
DW = D // 2
Q = 2048.0
NC = 2
NS = 16
NW = NC * NS
E_PER_W = N_EDGES // NW
CHUNK = 200
N_CHUNKS = E_PER_W // CHUNK
GRP = 8


def _norm_body(x_ref, o_ref):
    x = x_ref[...]
    n = jnp.sqrt(jnp.sum(x * x, axis=1, keepdims=True))
    o_ref[...] = x * (1.0 / jnp.maximum(n, 1e-8))


def _normalize(x):
    return pl.pallas_call(
        _norm_body,
        out_shape=jax.ShapeDtypeStruct((N_NODES, D), jnp.float32),
    )(x)


def _sc_edge_loss(xn, src, dst, tvec):
    mesh = plsc.VectorSubcoreMesh(core_axis_name="c", subcore_axis_name="s")

    @functools.partial(
        pl.kernel,
        out_type=jax.ShapeDtypeStruct((NW, 16), jnp.float32),
        mesh=mesh,
        compiler_params=pltpu.CompilerParams(use_tc_tiling_on_sc=False),
        scratch_types=[
            pltpu.VMEM((E_PER_W,), jnp.int32),
            pltpu.VMEM((E_PER_W,), jnp.int32),
            pltpu.VMEM((CHUNK, DW), jnp.int32),
            pltpu.VMEM((CHUNK, DW), jnp.int32),
            pltpu.VMEM((CHUNK, DW), jnp.int32),
            pltpu.VMEM((CHUNK, DW), jnp.int32),
            pltpu.VMEM((16,), jnp.float32),
            pltpu.SemaphoreType.DMA,
            pltpu.SemaphoreType.DMA,
            pltpu.SemaphoreType.DMA,
            pltpu.SemaphoreType.DMA,
        ],
    )
    def k(xn_hbm, src_hbm, dst_hbm, tv_hbm, out_hbm,
          src_v, dst_v, a0, a1, b0, b1, st_v, sa0, sa1, sb0, sb1):
        wid = lax.axis_index("s") * NC + lax.axis_index("c")
        base = wid * E_PER_W
        pltpu.sync_copy(src_hbm.at[pl.ds(base, E_PER_W)], src_v)
        pltpu.sync_copy(dst_hbm.at[pl.ds(base, E_PER_W)], dst_v)
        pltpu.sync_copy(tv_hbm, st_v)
        tv = st_v[...]
        lanes = lax.iota(jnp.int32, 16)
        rots = [(lanes + r) & 15 for r in (8, 4, 2, 1)]

        def issue(c, av, bv, sa, sb):
            pltpu.async_copy(xn_hbm.at[src_v.at[pl.ds(c * CHUNK, CHUNK)]], av, sa)
            pltpu.async_copy(xn_hbm.at[dst_v.at[pl.ds(c * CHUNK, CHUNK)]], bv, sb)

        def drain(av, bv, sa, sb):
            pltpu.make_async_copy(xn_hbm.at[pl.ds(0, CHUNK)], av, sa).wait()
            pltpu.make_async_copy(xn_hbm.at[pl.ds(0, CHUNK)], bv, sb).wait()

        def compute(av, bv, acc):
            def grp_body(i, acc2):
                e0 = i * GRP
                s = acc2
                for l in range(GRP):
                    e = e0 + l
                    vi = None
                    for j in range(DW // 16):
                        wa = av[e, pl.ds(16 * j, 16)]
                        wb = bv[e, pl.ds(16 * j, 16)]
                        a_lo = (wa << 16) >> 16
                        a_hi = wa >> 16
                        b_lo = (wb << 16) >> 16
                        b_hi = wb >> 16
                        p = a_lo * b_lo + a_hi * b_hi
                        vi = p if vi is None else vi + p
                    for r in rots:
                        vi = vi + _lane_take(vi, r)
                    v = vi.astype(jnp.float32) * jnp.float32(1.0 / (Q * Q))
                    s = s + jnp.maximum(tv - v, 0.0)
                return s

            return lax.fori_loop(0, CHUNK // GRP, grp_body, acc)

        issue(0, a0, b0, sa0, sb0)
        issue(1, a1, b1, sa1, sb1)

        def pair_body(i, acc):
            c0 = 2 * i
            drain(a0, b0, sa0, sb0)
            acc = compute(a0, b0, acc)

            @pl.when(c0 + 2 < N_CHUNKS)
            def _():
                issue(c0 + 2, a0, b0, sa0, sb0)

            drain(a1, b1, sa1, sb1)
            acc = compute(a1, b1, acc)

            @pl.when(c0 + 3 < N_CHUNKS)
            def _():
                issue(c0 + 3, a1, b1, sa1, sb1)

            return acc

        acc = lax.fori_loop(0, N_CHUNKS // 2, pair_body,
                            jnp.zeros((16,), jnp.float32))
        st_v[...] = acc * (1.0 / 16.0)
        pltpu.sync_copy(st_v, out_hbm.at[wid])

    return k(xn, src, dst, tvec)


def kernel(trigger_edge_index, x, thrd):
    ei = trigger_edge_index.astype(jnp.int32)
    xn = _normalize(x.astype(jnp.float32))
    q = jnp.round(xn * Q).astype(jnp.int32)
    packed = (q[:, 0::2] & 0xFFFF) | (q[:, 1::2] << 16)
    tvec = jnp.full((16,), thrd, dtype=jnp.float32)
    partials = _sc_edge_loss(packed, ei[0], ei[1], tvec)
    return jnp.sum(partials) * (1.0 / N_EDGES)


# speedup vs baseline: 4.0842x; 4.0842x over previous
import functools

import jax
import jax.numpy as jnp
from jax import lax
from jax.experimental import pallas as pl
from jax.experimental.pallas import tpu as pltpu
from jax.experimental.pallas import tpu_sc as plsc

def _lane_take(v, idx):
    dnums = lax.GatherDimensionNumbers(
        offset_dims=(), collapsed_slice_dims=(0,), start_index_map=(0,))
    return lax.gather(v, idx[:, None], dnums, slice_sizes=(1,),
                      mode=lax.GatherScatterMode.PROMISE_IN_BOUNDS)


N_NODES = 10000
N_EDGES = 320000
D = 128
DW = D // 2
Q = 2048.0
NC = 2
NS = 16
NW = NC * NS
E_PER_W = N_EDGES // NW
CHUNK = 200
N_CHUNKS = E_PER_W // CHUNK
GRP = 8


def _norm_body(x_ref, o_ref):
    x = x_ref[...]
    n = jnp.sqrt(jnp.sum(x * x, axis=1, keepdims=True))
    o_ref[...] = (x * (1.0 / jnp.maximum(n, 1e-8))).astype(jnp.bfloat16)


def _normalize(x):
    return pl.pallas_call(
        _norm_body,
        out_shape=jax.ShapeDtypeStruct((N_NODES, D), jnp.bfloat16),
    )(x)


def _sc_edge_loss(xn, src, dst, tvec):
    mesh = plsc.VectorSubcoreMesh(core_axis_name="c", subcore_axis_name="s")

    @functools.partial(
        pl.kernel,
        out_type=jax.ShapeDtypeStruct((NW, 16), jnp.float32),
        mesh=mesh,
        compiler_params=pltpu.CompilerParams(needs_layout_passes=False,
                                             use_tc_tiling_on_sc=False),
        scratch_types=[
            pltpu.VMEM((E_PER_W,), jnp.int32),
            pltpu.VMEM((E_PER_W,), jnp.int32),
            pltpu.VMEM((CHUNK, DW), jnp.int32),
            pltpu.VMEM((CHUNK, DW), jnp.int32),
            pltpu.VMEM((CHUNK, DW), jnp.int32),
            pltpu.VMEM((CHUNK, DW), jnp.int32),
            pltpu.VMEM((16,), jnp.float32),
            pltpu.SemaphoreType.DMA,
            pltpu.SemaphoreType.DMA,
            pltpu.SemaphoreType.DMA,
            pltpu.SemaphoreType.DMA,
        ],
    )
    def k(xn_hbm, src_hbm, dst_hbm, tv_hbm, out_hbm,
          src_v, dst_v, a0, a1, b0, b1, st_v, sa0, sa1, sb0, sb1):
        wid = lax.axis_index("s") * NC + lax.axis_index("c")
        base = wid * E_PER_W
        pltpu.sync_copy(src_hbm.at[pl.ds(base, E_PER_W)], src_v)
        pltpu.sync_copy(dst_hbm.at[pl.ds(base, E_PER_W)], dst_v)
        pltpu.sync_copy(tv_hbm, st_v)
        tv = st_v[...]
        lanes = lax.iota(jnp.int32, 16)
        rots = [(lanes + r) & 15 for r in (8, 4, 2, 1)]

        def issue(c, av, bv, sa, sb):
            pltpu.async_copy(xn_hbm.at[src_v.at[pl.ds(c * CHUNK, CHUNK)]], av, sa)
            pltpu.async_copy(xn_hbm.at[dst_v.at[pl.ds(c * CHUNK, CHUNK)]], bv, sb)

        def drain(av, bv, sa, sb):
            pltpu.make_async_copy(xn_hbm.at[pl.ds(0, CHUNK)], av, sa).wait()
            pltpu.make_async_copy(xn_hbm.at[pl.ds(0, CHUNK)], bv, sb).wait()

        def compute(av, bv, acc):
            def grp_body(i, acc2):
                e0 = i * GRP
                s = acc2
                for l in range(GRP):
                    e = e0 + l
                    prods = []
                    for j in range(DW // 16):
                        pa = plsc.bitcast(av[e, pl.ds(16 * j, 16)], jnp.bfloat16)
                        pb = plsc.bitcast(bv[e, pl.ds(16 * j, 16)], jnp.bfloat16)
                        prods.append(pa * pb)
                    p = (prods[0] + prods[1]) + (prods[2] + prods[3])
                    x1, x2 = plsc.unpack(p, format=plsc.PackFormat.INTERLEAVED)
                    v = x1 + x2
                    for r in rots:
                        v = v + _lane_take(v, r)
                    s = s + jnp.maximum(tv - v, 0.0)
                return s

            return lax.fori_loop(0, CHUNK // GRP, grp_body, acc)

        issue(0, a0, b0, sa0, sb0)
        issue(1, a1, b1, sa1, sb1)

        def pair_body(i, acc):
            c0 = 2 * i
            drain(a0, b0, sa0, sb0)
            acc = compute(a0, b0, acc)

            @pl.when(c0 + 2 < N_CHUNKS)
            def _():
                issue(c0 + 2, a0, b0, sa0, sb0)

            drain(a1, b1, sa1, sb1)
            acc = compute(a1, b1, acc)

            @pl.when(c0 + 3 < N_CHUNKS)
            def _():
                issue(c0 + 3, a1, b1, sa1, sb1)

            return acc

        acc = lax.fori_loop(0, N_CHUNKS // 2, pair_body,
                            jnp.zeros((16,), jnp.float32))
        st_v[...] = acc * (1.0 / 16.0)
        pltpu.sync_copy(st_v, out_hbm.at[wid])

    return k(xn, src, dst, tvec)


def kernel(trigger_edge_index, x, thrd):
    ei = trigger_edge_index.astype(jnp.int32)
    xn = _normalize(x.astype(jnp.float32))
    packed = lax.bitcast_convert_type(xn.reshape(N_NODES, DW, 2), jnp.int32)
    tvec = jnp.full((16,), thrd, dtype=jnp.float32)
    partials = _sc_edge_loss(packed, ei[0], ei[1], tvec)
    return jnp.sum(partials) * (1.0 / N_EDGES)


# speedup vs baseline: 4.0890x; 1.0012x over previous
import functools

import jax
import jax.numpy as jnp
from jax import lax
from jax.experimental import pallas as pl
from jax.experimental.pallas import tpu as pltpu
from jax.experimental.pallas import tpu_sc as plsc

def _lane_take(v, idx):
    dnums = lax.GatherDimensionNumbers(
        offset_dims=(), collapsed_slice_dims=(0,), start_index_map=(0,))
    return lax.gather(v, idx[:, None], dnums, slice_sizes=(1,),
                      mode=lax.GatherScatterMode.PROMISE_IN_BOUNDS)


N_NODES = 10000
N_EDGES = 320000
D = 128
DW = D // 2
Q = 2048.0
NC = 2
NS = 16
NW = NC * NS
E_PER_W = N_EDGES // NW
CHUNK = 200
N_CHUNKS = E_PER_W // CHUNK
GRP = 8


def _norm_body(x_ref, o_ref):
    x = x_ref[...]
    n = jnp.sqrt(jnp.sum(x * x, axis=1, keepdims=True))
    o_ref[...] = (x * (1.0 / jnp.maximum(n, 1e-8))).astype(jnp.bfloat16)


def _normalize(x):
    return pl.pallas_call(
        _norm_body,
        out_shape=jax.ShapeDtypeStruct((N_NODES, D), jnp.bfloat16),
    )(x)


def _sc_edge_loss(xn, src, dst, tvec):
    mesh = plsc.VectorSubcoreMesh(core_axis_name="c", subcore_axis_name="s")

    @functools.partial(
        pl.kernel,
        out_type=jax.ShapeDtypeStruct((NW, 16), jnp.float32),
        mesh=mesh,
        compiler_params=pltpu.CompilerParams(needs_layout_passes=False,
                                             use_tc_tiling_on_sc=False),
        scratch_types=[
            pltpu.VMEM((E_PER_W,), jnp.int32),
            pltpu.VMEM((E_PER_W,), jnp.int32),
            pltpu.VMEM((CHUNK, DW), jnp.int32),
            pltpu.VMEM((CHUNK, DW), jnp.int32),
            pltpu.VMEM((CHUNK, DW), jnp.int32),
            pltpu.VMEM((CHUNK, DW), jnp.int32),
            pltpu.VMEM((16,), jnp.float32),
            pltpu.SemaphoreType.DMA,
            pltpu.SemaphoreType.DMA,
            pltpu.SemaphoreType.DMA,
            pltpu.SemaphoreType.DMA,
        ],
    )
    def k(xn_hbm, src_hbm, dst_hbm, tv_hbm, out_hbm,
          src_v, dst_v, a0, a1, b0, b1, st_v, sa0, sa1, sb0, sb1):
        wid = lax.axis_index("c") * NS + lax.axis_index("s")
        base = wid * E_PER_W
        pltpu.sync_copy(src_hbm.at[pl.ds(base, E_PER_W)], src_v)
        pltpu.sync_copy(dst_hbm.at[pl.ds(base, E_PER_W)], dst_v)
        pltpu.sync_copy(tv_hbm, st_v)
        tv = st_v[...]
        lanes = lax.iota(jnp.int32, 16)
        rots = [(lanes + r) & 15 for r in (8, 4, 2, 1)]

        def issue(c, av, bv, sa, sb):
            pltpu.async_copy(xn_hbm.at[src_v.at[pl.ds(c * CHUNK, CHUNK)]], av, sa)
            pltpu.async_copy(xn_hbm.at[dst_v.at[pl.ds(c * CHUNK, CHUNK)]], bv, sb)

        def drain(av, bv, sa, sb):
            pltpu.make_async_copy(xn_hbm.at[pl.ds(0, CHUNK)], av, sa).wait()
            pltpu.make_async_copy(xn_hbm.at[pl.ds(0, CHUNK)], bv, sb).wait()

        def compute(av, bv, acc):
            def grp_body(i, acc2):
                e0 = i * GRP
                s = acc2
                for l in range(GRP):
                    e = e0 + l
                    prods = []
                    for j in range(DW // 16):
                        pa = plsc.bitcast(av[e, pl.ds(16 * j, 16)], jnp.bfloat16)
                        pb = plsc.bitcast(bv[e, pl.ds(16 * j, 16)], jnp.bfloat16)
                        prods.append(pa * pb)
                    p = (prods[0] + prods[1]) + (prods[2] + prods[3])
                    x1, x2 = plsc.unpack(p, format=plsc.PackFormat.INTERLEAVED)
                    v = x1 + x2
                    for r in rots:
                        v = v + _lane_take(v, r)
                    s = s + jnp.maximum(tv - v, 0.0)
                return s

            return lax.fori_loop(0, CHUNK // GRP, grp_body, acc)

        issue(0, a0, b0, sa0, sb0)
        issue(1, a1, b1, sa1, sb1)

        def pair_body(i, acc):
            c0 = 2 * i
            drain(a0, b0, sa0, sb0)
            acc = compute(a0, b0, acc)

            @pl.when(c0 + 2 < N_CHUNKS)
            def _():
                issue(c0 + 2, a0, b0, sa0, sb0)

            drain(a1, b1, sa1, sb1)
            acc = compute(a1, b1, acc)

            @pl.when(c0 + 3 < N_CHUNKS)
            def _():
                issue(c0 + 3, a1, b1, sa1, sb1)

            return acc

        acc = lax.fori_loop(0, N_CHUNKS // 2, pair_body,
                            jnp.zeros((16,), jnp.float32))
        st_v[...] = acc * (1.0 / 16.0)
        pltpu.sync_copy(st_v, out_hbm.at[wid])

    return k(xn, src, dst, tvec)


def kernel(trigger_edge_index, x, thrd):
    ei = trigger_edge_index.astype(jnp.int32)
    xn = _normalize(x.astype(jnp.float32))
    packed = lax.bitcast_convert_type(xn.reshape(N_NODES, DW, 2), jnp.int32)
    tvec = jnp.full((16,), thrd, dtype=jnp.float32)
    partials = _sc_edge_loss(packed, ei[0], ei[1], tvec)
    return jnp.sum(partials) * (1.0 / N_EDGES)
